# baseline (speedup 1.0000x reference)
import functools

import jax
import jax.numpy as jnp
from jax import lax
from jax.experimental import pallas as pl
from jax.experimental.pallas import tpu as pltpu
from jax.experimental.pallas import tpu_sc as plsc

_INFO = plsc.get_sparse_core_info()
_NC = _INFO.num_cores


def _sc_slice_body(B, S, in_hbm, out_hbm, buf):
    wid = lax.axis_index("s") * _NC + lax.axis_index("c")

    @pl.when(wid < B)
    def _():
        pltpu.sync_copy(in_hbm.at[wid, S - 1], buf)
        pltpu.sync_copy(buf, out_hbm.at[wid])


def kernel(inputs):
    B, S, D = inputs.shape
    mesh = plsc.VectorSubcoreMesh(core_axis_name="c", subcore_axis_name="s")
    k = pl.kernel(
        functools.partial(_sc_slice_body, B, S),
        mesh=mesh,
        out_type=jax.ShapeDtypeStruct((B, D), inputs.dtype),
        scratch_types=[pltpu.VMEM((D,), inputs.dtype)],
    )
    return k(inputs)
